# baseline (speedup 1.0000x reference)
import jax
import jax.numpy as jnp
import numpy as np
from jax.experimental import pallas as pl

DIM = 768
N_HEADS = 12
N_KV = 3
DH = DIM // N_HEADS
FF = int(DIM * 3.5)
R = 3
L = 2048
TQ = 256
_PATTERN = ['mod', 'mod', 'mor', 'plain', 'mod', 'mod', 'mor', 'plain']
_SCALE = 1.0 / float(np.sqrt(DH))


def _rms(x, w):
    return x * jax.lax.rsqrt(jnp.mean(x * x, axis=-1, keepdims=True) + 1e-6) * w


def _dot(a, b):
    return jnp.dot(a, b, preferred_element_type=jnp.float32)


def _qkv_kernel(x_ref, bias_ref, ln1_ref, wq_ref, wk_ref, wv_ref,
                q_ref, k_ref, v_ref):
    x = x_ref[...] + bias_ref[...]
    h = _rms(x, ln1_ref[...])
    for hd in range(N_HEADS):
        q_ref[hd] = _dot(h, wq_ref[hd])
    for g in range(N_KV):
        k_ref[g] = _dot(h, wk_ref[g])
        v_ref[g] = _dot(h, wv_ref[g])


def _attn_kernel(q_ref, k_ref, v_ref, o_ref):
    i = pl.program_id(1)
    q = q_ref[0]
    k = k_ref[0]
    s = jax.lax.dot_general(q, k, (((1,), (1,)), ((), ())),
                            preferred_element_type=jnp.float32) * _SCALE
    rows = jax.lax.broadcasted_iota(jnp.int32, (TQ, L), 0) + i * TQ
    cols = jax.lax.broadcasted_iota(jnp.int32, (TQ, L), 1)
    s = jnp.where(cols <= rows, s, jnp.float32(-1e30))
    m = jnp.max(s, axis=-1, keepdims=True)
    p = jnp.exp(s - m)
    w = p / jnp.sum(p, axis=-1, keepdims=True)
    o_ref[0] = _dot(w, v_ref[0])


def _post_body(xr_ref, bias_ref, o_ref, ln2_ref, wo_ref, w1_ref, w2_ref):
    xb = xr_ref[...] + bias_ref[...]
    acc = _dot(o_ref[0], wo_ref[0])
    for hd in range(1, N_HEADS):
        acc = acc + _dot(o_ref[hd], wo_ref[hd])
    x2 = xb + acc
    h2 = _rms(x2, ln2_ref[...])
    u = jax.nn.gelu(_dot(h2, w1_ref[...]))
    return xb, x2 + _dot(u, w2_ref[...])


def _post_plain_kernel(xr_ref, bias_ref, o_ref, ln2_ref, wo_ref, w1_ref,
                       w2_ref, out_ref):
    _, y = _post_body(xr_ref, bias_ref, o_ref, ln2_ref, wo_ref, w1_ref, w2_ref)
    out_ref[...] = y


def _post_mod_kernel(xr_ref, bias_ref, o_ref, ln2_ref, wo_ref, w1_ref,
                     w2_ref, wmod_ref, out_ref):
    xb, y = _post_body(xr_ref, bias_ref, o_ref, ln2_ref, wo_ref, w1_ref, w2_ref)
    g = jax.nn.sigmoid(jnp.sum(xb * wmod_ref[...], axis=-1, keepdims=True))
    out_ref[...] = g * y + (1.0 - g) * xb


def _blend_kernel(x0_ref, o0_ref, o1_ref, o2_ref, rw_ref, rb_ref, fln_ref,
                  out_ref):
    x0 = x0_ref[...]
    logits = jnp.clip(
        jnp.sum(x0 * rw_ref[...], axis=-1, keepdims=True) + rb_ref[0, 0],
        -3.0, 3.0)
    td = jax.nn.sigmoid(logits) * (R - 1)
    d0 = -(td - 0.0) ** 2
    d1 = -(td - 1.0) ** 2
    d2 = -(td - 2.0) ** 2
    m = jnp.maximum(d0, jnp.maximum(d1, d2))
    e0 = jnp.exp(d0 - m)
    e1 = jnp.exp(d1 - m)
    e2 = jnp.exp(d2 - m)
    s = e0 + e1 + e2
    out = (e0 * o0_ref[...] + e1 * o1_ref[...] + e2 * o2_ref[...]) / s
    out_ref[...] = _rms(out, fln_ref[...])


def _row_spec():
    return pl.BlockSpec((1, DIM), lambda *_: (0, 0))


def _seq_spec():
    return pl.BlockSpec((TQ, DIM), lambda i: (i, 0))


def _qkv(xs, bias, ln1, wq_r, wk_r, wv_r):
    return pl.pallas_call(
        _qkv_kernel,
        grid=(L // TQ,),
        in_specs=[
            _seq_spec(),
            _row_spec(),
            _row_spec(),
            pl.BlockSpec((N_HEADS, DIM, DH), lambda i: (0, 0, 0)),
            pl.BlockSpec((N_KV, DIM, DH), lambda i: (0, 0, 0)),
            pl.BlockSpec((N_KV, DIM, DH), lambda i: (0, 0, 0)),
        ],
        out_specs=[
            pl.BlockSpec((N_HEADS, TQ, DH), lambda i: (0, i, 0)),
            pl.BlockSpec((N_KV, TQ, DH), lambda i: (0, i, 0)),
            pl.BlockSpec((N_KV, TQ, DH), lambda i: (0, i, 0)),
        ],
        out_shape=[
            jax.ShapeDtypeStruct((N_HEADS, L, DH), jnp.float32),
            jax.ShapeDtypeStruct((N_KV, L, DH), jnp.float32),
            jax.ShapeDtypeStruct((N_KV, L, DH), jnp.float32),
        ],
    )(xs, bias, ln1, wq_r, wk_r, wv_r)


def _attn(q, k, v):
    return pl.pallas_call(
        _attn_kernel,
        grid=(N_HEADS, L // TQ),
        in_specs=[
            pl.BlockSpec((1, TQ, DH), lambda h, i: (h, i, 0)),
            pl.BlockSpec((1, L, DH), lambda h, i: (h // (N_HEADS // N_KV), 0, 0)),
            pl.BlockSpec((1, L, DH), lambda h, i: (h // (N_HEADS // N_KV), 0, 0)),
        ],
        out_specs=pl.BlockSpec((1, TQ, DH), lambda h, i: (h, i, 0)),
        out_shape=jax.ShapeDtypeStruct((N_HEADS, L, DH), jnp.float32),
    )(q, k, v)


def _post(xs, bias, o, ln2, wo_r, w1, w2, wmod=None):
    specs = [
        _seq_spec(),
        _row_spec(),
        pl.BlockSpec((N_HEADS, TQ, DH), lambda i: (0, i, 0)),
        _row_spec(),
        pl.BlockSpec((N_HEADS, DH, DIM), lambda i: (0, 0, 0)),
        pl.BlockSpec((DIM, FF), lambda i: (0, 0)),
        pl.BlockSpec((FF, DIM), lambda i: (0, 0)),
    ]
    args = [xs, bias, o, ln2, wo_r, w1, w2]
    body = _post_plain_kernel
    if wmod is not None:
        specs.append(_row_spec())
        args.append(wmod)
        body = _post_mod_kernel
    return pl.pallas_call(
        body,
        grid=(L // TQ,),
        in_specs=specs,
        out_specs=_seq_spec(),
        out_shape=jax.ShapeDtypeStruct((L, DIM), jnp.float32),
    )(*args)


def _blend(x0, outs, rw, rb, fln):
    return pl.pallas_call(
        _blend_kernel,
        grid=(L // TQ,),
        in_specs=[_seq_spec(), _seq_spec(), _seq_spec(), _seq_spec(),
                  _row_spec(), pl.BlockSpec((1, 1), lambda i: (0, 0)),
                  _row_spec()],
        out_specs=_seq_spec(),
        out_shape=jax.ShapeDtypeStruct((L, DIM), jnp.float32),
    )(x0, outs[0], outs[1], outs[2], rw, rb, fln)


def _attn_block(xs, bias, p):
    ln1 = p['ln1'].reshape(1, DIM)
    wq_r = p['wq'].reshape(DIM, N_HEADS, DH).transpose(1, 0, 2)
    wk_r = p['wk'].reshape(DIM, N_KV, DH).transpose(1, 0, 2)
    wv_r = p['wv'].reshape(DIM, N_KV, DH).transpose(1, 0, 2)
    wo_r = p['wo'].reshape(N_HEADS, DH, DIM)
    ln2 = p['ln2'].reshape(1, DIM)
    q, k, v = _qkv(xs, bias, ln1, wq_r, wk_r, wv_r)
    o = _attn(q, k, v)
    wmod = p['w_mod'].reshape(1, DIM) if 'w_mod' in p else None
    return _post(xs, bias, o, ln2, wo_r, p['w1'], p['w2'], wmod=wmod)


def kernel(x, params):
    xs = x.reshape(L, DIM)
    zero_bias = jnp.zeros((1, DIM), jnp.float32)
    for p, t in zip(params, _PATTERN):
        if t == 'mor':
            x0 = xs
            cur = xs
            outs = []
            for i in range(R):
                bias = (p['rec_embed'][i]
                        + p['rec_bias'][i].reshape(DIM)).reshape(1, DIM)
                pp = {k2: v2 for k2, v2 in p.items() if k2 != 'w_mod'}
                cur = _attn_block(cur, bias, pp)
                outs.append(cur)
            xs = _blend(x0, outs, p['rw'].reshape(1, DIM),
                        p['rb'].reshape(1, 1),
                        p['final_ln'].reshape(1, DIM))
        else:
            xs = _attn_block(xs, zero_bias, p)
    return xs.reshape(x.shape)
